# baseline (speedup 1.0000x reference)
import jax
import jax.numpy as jnp
from jax import lax
from jax.experimental import pallas as pl
from jax.experimental.pallas import tpu as pltpu
from jax.experimental.pallas import tpu_sc as plsc

NUM_CORES = 2
NUM_SUBCORES = 16
NUM_WORKERS = NUM_CORES * NUM_SUBCORES

BATCH = 16384
EMBED_DIM = 32
NUM_ROWS = 1000000
PACK = 128 // EMBED_DIM
TRANSPOSE_BN_ = 8192
_N_BLOCKS = (NUM_ROWS + TRANSPOSE_BN_ - 1) // TRANSPOSE_BN_
PACKED_ROWS = _N_BLOCKS * (TRANSPOSE_BN_ // PACK)
B_PER_W = BATCH // NUM_WORKERS
IDX_CHUNK = 128
N_IDX_CHUNKS = B_PER_W // IDX_CHUNK

TRANSPOSE_BN = TRANSPOSE_BN_


def _repack_body(tab_t_ref, out_ref):
  bn4 = TRANSPOSE_BN // PACK
  out_ref[...] = jnp.concatenate(
      [tab_t_ref[:, c * bn4:(c + 1) * bn4].T for c in range(PACK)], axis=1)


def _repack(tab_t):
  grid = (NUM_ROWS + TRANSPOSE_BN - 1) // TRANSPOSE_BN
  return pl.pallas_call(
      _repack_body,
      grid=(grid,),
      in_specs=[pl.BlockSpec((EMBED_DIM, TRANSPOSE_BN), lambda i: (0, i))],
      out_specs=pl.BlockSpec((TRANSPOSE_BN // PACK, 128), lambda i: (i, 0)),
      out_shape=jax.ShapeDtypeStruct((PACKED_ROWS, 128), jnp.float32),
  )(tab_t)


def _score_body(uid_hbm, pid_hbm, nid_hbm, ucg_hbm, pcg_hbm, ncg_hbm,
                utab_hbm, itab_hbm, pos_hbm, neg_hbm,
                uid_v, pid_v, nid_v, ucg_v, pcg_v, ncg_v,
                rows_v, pos_v, neg_v, sem):
  wid = lax.axis_index("s") * NUM_CORES + lax.axis_index("c")
  base = wid * B_PER_W

  for k in range(N_IDX_CHUNKS):
    off = pl.ds(base + k * IDX_CHUNK, IDX_CHUNK)
    pltpu.sync_copy(uid_hbm.at[off], uid_v.at[k])
    pltpu.sync_copy(pid_hbm.at[off], pid_v.at[k])
    pltpu.sync_copy(nid_hbm.at[off], nid_v.at[k])
    pltpu.sync_copy(ucg_hbm.at[off], ucg_v.at[k])
    pltpu.sync_copy(pcg_hbm.at[off], pcg_v.at[k])
    pltpu.sync_copy(ncg_hbm.at[off], ncg_v.at[k])

  def fire(k, buf):
    pltpu.async_copy(utab_hbm.at[uid_v.at[k]], rows_v.at[buf, 0], sem)
    pltpu.async_copy(itab_hbm.at[pid_v.at[k]], rows_v.at[buf, 1], sem)
    pltpu.async_copy(itab_hbm.at[nid_v.at[k]], rows_v.at[buf, 2], sem)

  def drain(buf):
    for r in range(3):
      pltpu.make_async_copy(
          utab_hbm.at[pl.ds(0, IDX_CHUNK)], rows_v.at[buf, r], sem).wait()

  lane = lax.iota(jnp.int32, 16)

  fire(0, 0)
  for k in range(N_IDX_CHUNKS):
    buf = k % 2
    drain(buf)
    if k + 1 < N_IDX_CHUNKS:
      fire(k + 1, (k + 1) % 2)

    def group(g, carry):
      rows = g * 16 + lane
      cu = EMBED_DIM * ucg_v[k, pl.ds(g * 16, 16)]
      cp = EMBED_DIM * pcg_v[k, pl.ds(g * 16, 16)]
      cn = EMBED_DIM * ncg_v[k, pl.ds(g * 16, 16)]
      accp = jnp.zeros((16,), jnp.float32)
      accn = jnp.zeros((16,), jnp.float32)
      for f in range(EMBED_DIM):
        u = plsc.load_gather(rows_v.at[buf, 0], [rows, cu + f])
        p = plsc.load_gather(rows_v.at[buf, 1], [rows, cp + f])
        n = plsc.load_gather(rows_v.at[buf, 2], [rows, cn + f])
        accp = accp + u * p
        accn = accn + u * n
      out = pl.ds(k * IDX_CHUNK + g * 16, 16)
      pos_v[out] = accp
      neg_v[out] = accn
      return carry

    lax.fori_loop(0, IDX_CHUNK // 16, group, 0)

  pltpu.sync_copy(pos_v, pos_hbm.at[pl.ds(base, B_PER_W)])
  pltpu.sync_copy(neg_v, neg_hbm.at[pl.ds(base, B_PER_W)])


def _sc_score(uid4, pid4, nid4, ucg, pcg, ncg, utab, itab):
  mesh = plsc.VectorSubcoreMesh(core_axis_name="c", subcore_axis_name="s")
  idx_t = pltpu.VMEM((N_IDX_CHUNKS, IDX_CHUNK), jnp.int32)
  f = pl.kernel(
      _score_body,
      out_type=(
          jax.ShapeDtypeStruct((BATCH,), jnp.float32),
          jax.ShapeDtypeStruct((BATCH,), jnp.float32),
      ),
      mesh=mesh,
      scratch_types=(
          idx_t, idx_t, idx_t, idx_t, idx_t, idx_t,
          pltpu.VMEM((2, 3, IDX_CHUNK, 128), jnp.float32),
          pltpu.VMEM((B_PER_W,), jnp.float32),
          pltpu.VMEM((B_PER_W,), jnp.float32),
          pltpu.SemaphoreType.DMA,
      ),
      compiler_params=pltpu.CompilerParams(
          needs_layout_passes=False, use_tc_tiling_on_sc=True),
  )
  return f(uid4, pid4, nid4, ucg, pcg, ncg, utab, itab)


@jax.jit
def kernel(user_ids, pos_item_ids, neg_item_ids, user_table, item_table):
  user_ids = user_ids.astype(jnp.int32)
  pos_item_ids = pos_item_ids.astype(jnp.int32)
  neg_item_ids = neg_item_ids.astype(jnp.int32)

  utab = _repack(user_table.T)
  itab = _repack(item_table.T)

  bn4 = TRANSPOSE_BN // PACK
  def packed_row(u):
    return (u // TRANSPOSE_BN) * bn4 + u % bn4
  def lane_group(u):
    return (u // bn4) % PACK

  return _sc_score(
      packed_row(user_ids), packed_row(pos_item_ids),
      packed_row(neg_item_ids),
      lane_group(user_ids), lane_group(pos_item_ids),
      lane_group(neg_item_ids), utab, itab)


# speedup vs baseline: 1.3695x; 1.3695x over previous
import jax
import jax.numpy as jnp
from jax import lax
from jax.experimental import pallas as pl
from jax.experimental.pallas import tpu as pltpu
from jax.experimental.pallas import tpu_sc as plsc

NUM_CORES = 2
NUM_SUBCORES = 16
NUM_WORKERS = NUM_CORES * NUM_SUBCORES

BATCH = 16384
EMBED_DIM = 32
NUM_ROWS = 1000000
PACK = 128 // EMBED_DIM
TRANSPOSE_BN_ = 8192
_N_BLOCKS = (NUM_ROWS + TRANSPOSE_BN_ - 1) // TRANSPOSE_BN_
PACKED_ROWS = _N_BLOCKS * (TRANSPOSE_BN_ // PACK)
B_PER_W = BATCH // NUM_WORKERS
IDX_CHUNK = 128
N_IDX_CHUNKS = B_PER_W // IDX_CHUNK

TRANSPOSE_BN = TRANSPOSE_BN_


def _repack_body(tab_t_ref, out_ref):
  bn4 = TRANSPOSE_BN // PACK
  x = tab_t_ref[...].astype(jnp.bfloat16)
  out_ref[...] = jnp.concatenate(
      [x[:, c * bn4:(c + 1) * bn4].T for c in range(PACK)],
      axis=1).astype(jnp.float32)


def _repack(tab_t):
  grid = (NUM_ROWS + TRANSPOSE_BN - 1) // TRANSPOSE_BN
  return pl.pallas_call(
      _repack_body,
      grid=(grid,),
      in_specs=[pl.BlockSpec((EMBED_DIM, TRANSPOSE_BN), lambda i: (0, i))],
      out_specs=pl.BlockSpec((TRANSPOSE_BN // PACK, 128), lambda i: (i, 0)),
      out_shape=jax.ShapeDtypeStruct((PACKED_ROWS, 128), jnp.float32),
  )(tab_t)


def _score_body(uid_hbm, pid_hbm, nid_hbm, ucg_hbm, pcg_hbm, ncg_hbm,
                utab_hbm, itab_hbm, pos_hbm, neg_hbm,
                uid_v, pid_v, nid_v, ucg_v, pcg_v, ncg_v,
                rows_v, pos_v, neg_v, sem):
  wid = lax.axis_index("s") * NUM_CORES + lax.axis_index("c")
  base = wid * B_PER_W

  for k in range(N_IDX_CHUNKS):
    off = pl.ds(base + k * IDX_CHUNK, IDX_CHUNK)
    pltpu.sync_copy(uid_hbm.at[off], uid_v.at[k])
    pltpu.sync_copy(pid_hbm.at[off], pid_v.at[k])
    pltpu.sync_copy(nid_hbm.at[off], nid_v.at[k])
    pltpu.sync_copy(ucg_hbm.at[off], ucg_v.at[k])
    pltpu.sync_copy(pcg_hbm.at[off], pcg_v.at[k])
    pltpu.sync_copy(ncg_hbm.at[off], ncg_v.at[k])

  def fire(k, buf):
    pltpu.async_copy(utab_hbm.at[uid_v.at[k]], rows_v.at[buf, 0], sem)
    pltpu.async_copy(itab_hbm.at[pid_v.at[k]], rows_v.at[buf, 1], sem)
    pltpu.async_copy(itab_hbm.at[nid_v.at[k]], rows_v.at[buf, 2], sem)

  def drain(buf):
    for r in range(3):
      pltpu.make_async_copy(
          utab_hbm.at[pl.ds(0, IDX_CHUNK)], rows_v.at[buf, r], sem).wait()

  lane = lax.iota(jnp.int32, 16)

  fire(0, 0)
  for k in range(N_IDX_CHUNKS):
    buf = k % 2
    drain(buf)
    if k + 1 < N_IDX_CHUNKS:
      fire(k + 1, (k + 1) % 2)

    def group(g, carry):
      rows = g * 16 + lane
      cu = EMBED_DIM * ucg_v[k, pl.ds(g * 16, 16)]
      cp = EMBED_DIM * pcg_v[k, pl.ds(g * 16, 16)]
      cn = EMBED_DIM * ncg_v[k, pl.ds(g * 16, 16)]
      accp = jnp.zeros((16,), jnp.float32)
      accn = jnp.zeros((16,), jnp.float32)
      for f in range(EMBED_DIM):
        u = plsc.load_gather(rows_v.at[buf, 0], [rows, cu + f])
        p = plsc.load_gather(rows_v.at[buf, 1], [rows, cp + f])
        n = plsc.load_gather(rows_v.at[buf, 2], [rows, cn + f])
        accp = accp + u * p
        accn = accn + u * n
      out = pl.ds(k * IDX_CHUNK + g * 16, 16)
      pos_v[out] = accp
      neg_v[out] = accn
      return carry

    lax.fori_loop(0, IDX_CHUNK // 16, group, 0)

  pltpu.sync_copy(pos_v, pos_hbm.at[pl.ds(base, B_PER_W)])
  pltpu.sync_copy(neg_v, neg_hbm.at[pl.ds(base, B_PER_W)])


def _sc_score(uid4, pid4, nid4, ucg, pcg, ncg, utab, itab):
  mesh = plsc.VectorSubcoreMesh(core_axis_name="c", subcore_axis_name="s")
  idx_t = pltpu.VMEM((N_IDX_CHUNKS, IDX_CHUNK), jnp.int32)
  f = pl.kernel(
      _score_body,
      out_type=(
          jax.ShapeDtypeStruct((BATCH,), jnp.float32),
          jax.ShapeDtypeStruct((BATCH,), jnp.float32),
      ),
      mesh=mesh,
      scratch_types=(
          idx_t, idx_t, idx_t, idx_t, idx_t, idx_t,
          pltpu.VMEM((2, 3, IDX_CHUNK, 128), jnp.float32),
          pltpu.VMEM((B_PER_W,), jnp.float32),
          pltpu.VMEM((B_PER_W,), jnp.float32),
          pltpu.SemaphoreType.DMA,
      ),
      compiler_params=pltpu.CompilerParams(
          needs_layout_passes=False, use_tc_tiling_on_sc=True),
  )
  return f(uid4, pid4, nid4, ucg, pcg, ncg, utab, itab)


@jax.jit
def kernel(user_ids, pos_item_ids, neg_item_ids, user_table, item_table):
  user_ids = user_ids.astype(jnp.int32)
  pos_item_ids = pos_item_ids.astype(jnp.int32)
  neg_item_ids = neg_item_ids.astype(jnp.int32)

  utab = _repack(user_table.T)
  itab = _repack(item_table.T)

  bn4 = TRANSPOSE_BN // PACK
  def packed_row(u):
    return (u // TRANSPOSE_BN) * bn4 + u % bn4
  def lane_group(u):
    return (u // bn4) % PACK

  return _sc_score(
      packed_row(user_ids), packed_row(pos_item_ids),
      packed_row(neg_item_ids),
      lane_group(user_ids), lane_group(pos_item_ids),
      lane_group(neg_item_ids), utab, itab)


# speedup vs baseline: 1.6367x; 1.1951x over previous
import jax
import jax.numpy as jnp
from jax import lax
from jax.experimental import pallas as pl
from jax.experimental.pallas import tpu as pltpu
from jax.experimental.pallas import tpu_sc as plsc

NUM_CORES = 2
NUM_SUBCORES = 16
NUM_WORKERS = NUM_CORES * NUM_SUBCORES

BATCH = 16384
EMBED_DIM = 32
NUM_ROWS = 1000000
PACK = 128 // EMBED_DIM
TRANSPOSE_BN_ = 16384
_N_BLOCKS = (NUM_ROWS + TRANSPOSE_BN_ - 1) // TRANSPOSE_BN_
PACKED_ROWS = _N_BLOCKS * (TRANSPOSE_BN_ // PACK)
B_PER_W = BATCH // NUM_WORKERS
IDX_CHUNK = 128
N_IDX_CHUNKS = B_PER_W // IDX_CHUNK

TRANSPOSE_BN = TRANSPOSE_BN_


def _repack_body(tab_t_ref, out_ref):
  bn4 = TRANSPOSE_BN // PACK
  x = tab_t_ref[...].astype(jnp.bfloat16)
  out_ref[...] = jnp.concatenate(
      [x[:, c * bn4:(c + 1) * bn4].T for c in range(PACK)],
      axis=1).astype(jnp.float32)


def _repack(tab_t):
  grid = (NUM_ROWS + TRANSPOSE_BN - 1) // TRANSPOSE_BN
  return pl.pallas_call(
      _repack_body,
      grid=(grid,),
      in_specs=[pl.BlockSpec((EMBED_DIM, TRANSPOSE_BN), lambda i: (0, i))],
      out_specs=pl.BlockSpec((TRANSPOSE_BN // PACK, 128), lambda i: (i, 0)),
      out_shape=jax.ShapeDtypeStruct((PACKED_ROWS, 128), jnp.float32),
  )(tab_t)


def _score_body(uid_hbm, pid_hbm, nid_hbm, ucg_hbm, pcg_hbm, ncg_hbm,
                utab_hbm, itab_hbm, pos_hbm, neg_hbm,
                uid_v, pid_v, nid_v, ucg_v, pcg_v, ncg_v,
                rows_v, pos_v, neg_v, sem):
  wid = lax.axis_index("s") * NUM_CORES + lax.axis_index("c")
  base = wid * B_PER_W

  for k in range(N_IDX_CHUNKS):
    off = pl.ds(base + k * IDX_CHUNK, IDX_CHUNK)
    pltpu.sync_copy(uid_hbm.at[off], uid_v.at[k])
    pltpu.sync_copy(pid_hbm.at[off], pid_v.at[k])
    pltpu.sync_copy(nid_hbm.at[off], nid_v.at[k])
    pltpu.sync_copy(ucg_hbm.at[off], ucg_v.at[k])
    pltpu.sync_copy(pcg_hbm.at[off], pcg_v.at[k])
    pltpu.sync_copy(ncg_hbm.at[off], ncg_v.at[k])

  def fire(k, buf):
    pltpu.async_copy(utab_hbm.at[uid_v.at[k]], rows_v.at[buf, 0], sem)
    pltpu.async_copy(itab_hbm.at[pid_v.at[k]], rows_v.at[buf, 1], sem)
    pltpu.async_copy(itab_hbm.at[nid_v.at[k]], rows_v.at[buf, 2], sem)

  def drain(buf):
    for r in range(3):
      pltpu.make_async_copy(
          utab_hbm.at[pl.ds(0, IDX_CHUNK)], rows_v.at[buf, r], sem).wait()

  lane = lax.iota(jnp.int32, 16)

  fire(0, 0)
  for k in range(N_IDX_CHUNKS):
    buf = k % 2
    drain(buf)
    if k + 1 < N_IDX_CHUNKS:
      fire(k + 1, (k + 1) % 2)

    def group(g, carry):
      rows = g * 16 + lane
      cu = EMBED_DIM * ucg_v[k, pl.ds(g * 16, 16)]
      cp = EMBED_DIM * pcg_v[k, pl.ds(g * 16, 16)]
      cn = EMBED_DIM * ncg_v[k, pl.ds(g * 16, 16)]
      accp = jnp.zeros((16,), jnp.float32)
      accn = jnp.zeros((16,), jnp.float32)
      for f in range(EMBED_DIM):
        u = plsc.load_gather(rows_v.at[buf, 0], [rows, cu + f])
        p = plsc.load_gather(rows_v.at[buf, 1], [rows, cp + f])
        n = plsc.load_gather(rows_v.at[buf, 2], [rows, cn + f])
        accp = accp + u * p
        accn = accn + u * n
      out = pl.ds(k * IDX_CHUNK + g * 16, 16)
      pos_v[out] = accp
      neg_v[out] = accn
      return carry

    lax.fori_loop(0, IDX_CHUNK // 16, group, 0)

  pltpu.sync_copy(pos_v, pos_hbm.at[pl.ds(base, B_PER_W)])
  pltpu.sync_copy(neg_v, neg_hbm.at[pl.ds(base, B_PER_W)])


def _sc_score(uid4, pid4, nid4, ucg, pcg, ncg, utab, itab):
  mesh = plsc.VectorSubcoreMesh(core_axis_name="c", subcore_axis_name="s")
  idx_t = pltpu.VMEM((N_IDX_CHUNKS, IDX_CHUNK), jnp.int32)
  f = pl.kernel(
      _score_body,
      out_type=(
          jax.ShapeDtypeStruct((BATCH,), jnp.float32),
          jax.ShapeDtypeStruct((BATCH,), jnp.float32),
      ),
      mesh=mesh,
      scratch_types=(
          idx_t, idx_t, idx_t, idx_t, idx_t, idx_t,
          pltpu.VMEM((2, 3, IDX_CHUNK, 128), jnp.float32),
          pltpu.VMEM((B_PER_W,), jnp.float32),
          pltpu.VMEM((B_PER_W,), jnp.float32),
          pltpu.SemaphoreType.DMA,
      ),
      compiler_params=pltpu.CompilerParams(
          needs_layout_passes=False, use_tc_tiling_on_sc=True),
  )
  return f(uid4, pid4, nid4, ucg, pcg, ncg, utab, itab)


@jax.jit
def kernel(user_ids, pos_item_ids, neg_item_ids, user_table, item_table):
  user_ids = user_ids.astype(jnp.int32)
  pos_item_ids = pos_item_ids.astype(jnp.int32)
  neg_item_ids = neg_item_ids.astype(jnp.int32)

  utab = _repack(user_table.T)
  itab = _repack(item_table.T)

  bn4 = TRANSPOSE_BN // PACK
  def packed_row(u):
    return (u // TRANSPOSE_BN) * bn4 + u % bn4
  def lane_group(u):
    return (u // bn4) % PACK

  return _sc_score(
      packed_row(user_ids), packed_row(pos_item_ids),
      packed_row(neg_item_ids),
      lane_group(user_ids), lane_group(pos_item_ids),
      lane_group(neg_item_ids), utab, itab)


# speedup vs baseline: 1.7885x; 1.0927x over previous
import jax
import jax.numpy as jnp
from jax import lax
from jax.experimental import pallas as pl
from jax.experimental.pallas import tpu as pltpu
from jax.experimental.pallas import tpu_sc as plsc

NUM_CORES = 2
NUM_SUBCORES = 16
NUM_WORKERS = NUM_CORES * NUM_SUBCORES

BATCH = 16384
EMBED_DIM = 32
NUM_ROWS = 1000000
PACK = 128 // EMBED_DIM
TRANSPOSE_BN_ = 32768
_N_BLOCKS = (NUM_ROWS + TRANSPOSE_BN_ - 1) // TRANSPOSE_BN_
PACKED_ROWS = _N_BLOCKS * (TRANSPOSE_BN_ // PACK)
B_PER_W = BATCH // NUM_WORKERS
IDX_CHUNK = 128
N_IDX_CHUNKS = B_PER_W // IDX_CHUNK

TRANSPOSE_BN = TRANSPOSE_BN_


def _repack_body(tab_t_ref, out_ref):
  bn4 = TRANSPOSE_BN // PACK
  x = tab_t_ref[...].astype(jnp.bfloat16)
  out_ref[...] = jnp.concatenate(
      [x[:, c * bn4:(c + 1) * bn4].T for c in range(PACK)],
      axis=1).astype(jnp.float32)


def _repack(tab_t):
  grid = (NUM_ROWS + TRANSPOSE_BN - 1) // TRANSPOSE_BN
  return pl.pallas_call(
      _repack_body,
      grid=(grid,),
      in_specs=[pl.BlockSpec((EMBED_DIM, TRANSPOSE_BN), lambda i: (0, i))],
      out_specs=pl.BlockSpec((TRANSPOSE_BN // PACK, 128), lambda i: (i, 0)),
      out_shape=jax.ShapeDtypeStruct((PACKED_ROWS, 128), jnp.float32),
  )(tab_t)


def _score_body(uid_hbm, pid_hbm, nid_hbm, ucg_hbm, pcg_hbm, ncg_hbm,
                utab_hbm, itab_hbm, pos_hbm, neg_hbm,
                uid_v, pid_v, nid_v, ucg_v, pcg_v, ncg_v,
                rows_v, pos_v, neg_v, sem):
  wid = lax.axis_index("s") * NUM_CORES + lax.axis_index("c")
  base = wid * B_PER_W

  for k in range(N_IDX_CHUNKS):
    off = pl.ds(base + k * IDX_CHUNK, IDX_CHUNK)
    pltpu.sync_copy(uid_hbm.at[off], uid_v.at[k])
    pltpu.sync_copy(pid_hbm.at[off], pid_v.at[k])
    pltpu.sync_copy(nid_hbm.at[off], nid_v.at[k])
    pltpu.sync_copy(ucg_hbm.at[off], ucg_v.at[k])
    pltpu.sync_copy(pcg_hbm.at[off], pcg_v.at[k])
    pltpu.sync_copy(ncg_hbm.at[off], ncg_v.at[k])

  def fire(k, buf):
    pltpu.async_copy(utab_hbm.at[uid_v.at[k]], rows_v.at[buf, 0], sem)
    pltpu.async_copy(itab_hbm.at[pid_v.at[k]], rows_v.at[buf, 1], sem)
    pltpu.async_copy(itab_hbm.at[nid_v.at[k]], rows_v.at[buf, 2], sem)

  def drain(buf):
    for r in range(3):
      pltpu.make_async_copy(
          utab_hbm.at[pl.ds(0, IDX_CHUNK)], rows_v.at[buf, r], sem).wait()

  lane = lax.iota(jnp.int32, 16)

  fire(0, 0)
  for k in range(N_IDX_CHUNKS):
    buf = k % 2
    drain(buf)
    if k + 1 < N_IDX_CHUNKS:
      fire(k + 1, (k + 1) % 2)

    def group(g, carry):
      rows = g * 16 + lane
      cu = EMBED_DIM * ucg_v[k, pl.ds(g * 16, 16)]
      cp = EMBED_DIM * pcg_v[k, pl.ds(g * 16, 16)]
      cn = EMBED_DIM * ncg_v[k, pl.ds(g * 16, 16)]
      accp = jnp.zeros((16,), jnp.float32)
      accn = jnp.zeros((16,), jnp.float32)
      for f in range(EMBED_DIM):
        u = plsc.load_gather(rows_v.at[buf, 0], [rows, cu + f])
        p = plsc.load_gather(rows_v.at[buf, 1], [rows, cp + f])
        n = plsc.load_gather(rows_v.at[buf, 2], [rows, cn + f])
        accp = accp + u * p
        accn = accn + u * n
      out = pl.ds(k * IDX_CHUNK + g * 16, 16)
      pos_v[out] = accp
      neg_v[out] = accn
      return carry

    lax.fori_loop(0, IDX_CHUNK // 16, group, 0)

  pltpu.sync_copy(pos_v, pos_hbm.at[pl.ds(base, B_PER_W)])
  pltpu.sync_copy(neg_v, neg_hbm.at[pl.ds(base, B_PER_W)])


def _sc_score(uid4, pid4, nid4, ucg, pcg, ncg, utab, itab):
  mesh = plsc.VectorSubcoreMesh(core_axis_name="c", subcore_axis_name="s")
  idx_t = pltpu.VMEM((N_IDX_CHUNKS, IDX_CHUNK), jnp.int32)
  f = pl.kernel(
      _score_body,
      out_type=(
          jax.ShapeDtypeStruct((BATCH,), jnp.float32),
          jax.ShapeDtypeStruct((BATCH,), jnp.float32),
      ),
      mesh=mesh,
      scratch_types=(
          idx_t, idx_t, idx_t, idx_t, idx_t, idx_t,
          pltpu.VMEM((2, 3, IDX_CHUNK, 128), jnp.float32),
          pltpu.VMEM((B_PER_W,), jnp.float32),
          pltpu.VMEM((B_PER_W,), jnp.float32),
          pltpu.SemaphoreType.DMA,
      ),
      compiler_params=pltpu.CompilerParams(
          needs_layout_passes=False, use_tc_tiling_on_sc=True),
  )
  return f(uid4, pid4, nid4, ucg, pcg, ncg, utab, itab)


@jax.jit
def kernel(user_ids, pos_item_ids, neg_item_ids, user_table, item_table):
  user_ids = user_ids.astype(jnp.int32)
  pos_item_ids = pos_item_ids.astype(jnp.int32)
  neg_item_ids = neg_item_ids.astype(jnp.int32)

  utab = _repack(user_table.T)
  itab = _repack(item_table.T)

  bn4 = TRANSPOSE_BN // PACK
  def packed_row(u):
    return (u // TRANSPOSE_BN) * bn4 + u % bn4
  def lane_group(u):
    return (u // bn4) % PACK

  return _sc_score(
      packed_row(user_ids), packed_row(pos_item_ids),
      packed_row(neg_item_ids),
      lane_group(user_ids), lane_group(pos_item_ids),
      lane_group(neg_item_ids), utab, itab)
